# baseline (speedup 1.0000x reference)
import jax
import jax.numpy as jnp
from jax import lax
from jax.experimental import pallas as pl
from jax.experimental.pallas import tpu as pltpu
from jax.experimental.pallas import tpu_sc as plsc

NC = 2
NS = 16
NW = NC * NS
CHUNK = 125


def _mesh():
    return plsc.VectorSubcoreMesh(
        core_axis_name="c", subcore_axis_name="s", num_cores=NC, num_subcores=NS)


_SC_PARAMS = pltpu.CompilerParams(
    use_tc_tiling_on_sc=False, needs_layout_passes=False)


def _sc_gather(table, idx2):
    n_rows = idx2.shape[0]
    ch = n_rows // NW
    epw = ch * CHUNK
    feat = table.shape[1]
    kk = 8

    def body(table_hbm, idx_hbm, out_hbm, idx_v, rows_v, sem):
        wid = lax.axis_index("c") * NS + lax.axis_index("s")
        rowbase = wid * ch
        base = wid * epw
        pltpu.sync_copy(idx_hbm.at[pl.ds(rowbase, ch)], idx_v)

        def group(g, carry):
            c0 = g * kk
            handles = [
                pltpu.async_copy(table_hbm.at[idx_v.at[c0 + j]],
                                 rows_v.at[pl.ds((c0 + j) * CHUNK, CHUNK)], sem)
                for j in range(kk)
            ]
            for hd in handles:
                hd.wait()
            return carry

        lax.fori_loop(0, ch // kk, group, 0)
        pltpu.sync_copy(rows_v, out_hbm.at[pl.ds(base, epw)])

    return pl.kernel(
        body,
        out_type=jax.ShapeDtypeStruct((n_rows * CHUNK, feat), jnp.float32),
        mesh=_mesh(),
        compiler_params=_SC_PARAMS,
        scratch_types=[
            pltpu.VMEM((ch, CHUNK), jnp.int32),
            pltpu.VMEM((epw, feat), jnp.float32),
            pltpu.SemaphoreType.DMA,
        ],
    )(table, idx2)


def _sc_mega0(m, dst2, src2, nn_bias, gamma, beta, n_nodes):
    e, feat = m.shape
    et = e // NS
    hch = et // 2 // CHUNK
    gch = e // NW // CHUNK
    gpw = gch * CHUNK
    rps = n_nodes // NS
    kk = 8
    inv_n = 1.0 / n_nodes

    def body(m_hbm, dsti_hbm, srci_hbm, bias_hbm, gam_hbm, bet_hbm,
             hs1_hbm, idx_v, m_v, work_v, stats_my, stats_all, pvec,
             acc, stats_sh, sem):
        cid = lax.axis_index("c")
        sid = lax.axis_index("s")
        wid = cid * NS + sid

        def zb(i, c0):
            work_v[i, :] = jnp.zeros((feat,), jnp.float32)
            return c0

        lax.fori_loop(0, rps, zb, 0)
        pltpu.sync_copy(work_v, acc.at[pl.ds(sid * rps, rps)])
        pltpu.sync_copy(bias_hbm, pvec.at[0])
        pltpu.sync_copy(gam_hbm, pvec.at[1])
        pltpu.sync_copy(bet_hbm, pvec.at[2])
        pltpu.sync_copy(dsti_hbm.at[pl.ds(sid * 2 * hch, 2 * hch)], idx_v)
        plsc.subcore_barrier()

        for half in range(2):
            pltpu.sync_copy(
                m_hbm.at[pl.ds(sid * et + half * hch * CHUNK, hch * CHUNK)], m_v)

            def grp(g, c0, _half=half):
                cb = g * kk
                handles = [
                    pltpu.async_copy(m_v.at[pl.ds((cb + j) * CHUNK, CHUNK)],
                                     acc.at[idx_v.at[_half * hch + cb + j]],
                                     sem, add=True)
                    for j in range(kk)
                ]
                for hd in handles:
                    hd.wait()
                return c0

            lax.fori_loop(0, hch // kk, grp, 0)
        plsc.subcore_barrier()

        pltpu.sync_copy(acc.at[pl.ds(sid * rps, rps)], work_v)
        bias_v = pvec[0, :]
        gam_v = pvec[1, :]
        bet_v = pvec[2, :]

        def bn1(i, carry):
            s, ss = carry
            v = jnp.maximum(work_v[i, :] + bias_v, 0.0)
            work_v[i, :] = v
            return (s + v, ss + v * v)

        zv = jnp.zeros((feat,), jnp.float32)
        s, ss = lax.fori_loop(0, rps, bn1, (zv, zv))
        stats_my[pl.ds(0, feat)] = s
        stats_my[pl.ds(feat, feat)] = ss
        pltpu.sync_copy(stats_my, stats_sh.at[sid])
        plsc.subcore_barrier()

        pltpu.sync_copy(stats_sh, stats_all)
        tot = jnp.zeros((feat,), jnp.float32)
        tot2 = jnp.zeros((feat,), jnp.float32)
        for t in range(NS):
            tot = tot + stats_all[t, pl.ds(0, feat)]
            tot2 = tot2 + stats_all[t, pl.ds(feat, feat)]
        mean = tot * inv_n
        var = tot2 * inv_n - mean * mean
        x = var + 1e-5
        yi = jnp.int32(0x5F3759DF) - (plsc.bitcast(x, jnp.int32) >> 1)
        y = plsc.bitcast(yi, jnp.float32)
        for _ in range(3):
            y = y * (1.5 - 0.5 * x * y * y)
        scale = gam_v * y
        shift = bet_v - mean * scale

        def bn2(i, c0):
            work_v[i, :] = work_v[i, :] * scale + shift
            return c0

        lax.fori_loop(0, rps, bn2, 0)
        pltpu.sync_copy(work_v, acc.at[pl.ds(sid * rps, rps)])
        plsc.subcore_barrier()

        pltpu.sync_copy(srci_hbm.at[pl.ds(wid * gch, gch)],
                        idx_v.at[pl.ds(0, gch)])

        def ggrp(g, c0):
            cb = g * kk
            handles = [
                pltpu.async_copy(acc.at[idx_v.at[cb + j]],
                                 m_v.at[pl.ds((cb + j) * CHUNK, CHUNK)], sem)
                for j in range(kk)
            ]
            for hd in handles:
                hd.wait()
            return c0

        lax.fori_loop(0, gch // kk, ggrp, 0)
        pltpu.sync_copy(m_v, hs1_hbm.at[pl.ds(wid * gpw, gpw)])

    return pl.kernel(
        body,
        out_type=jax.ShapeDtypeStruct((e, feat), jnp.float32),
        mesh=_mesh(),
        compiler_params=_SC_PARAMS,
        scratch_types=[
            pltpu.VMEM((2 * hch, CHUNK), jnp.int32),
            pltpu.VMEM((hch * CHUNK, feat), jnp.float32),
            pltpu.VMEM((rps, feat), jnp.float32),
            pltpu.VMEM((2 * feat,), jnp.float32),
            pltpu.VMEM((NS, 2 * feat), jnp.float32),
            pltpu.VMEM((3, feat), jnp.float32),
            pltpu.VMEM_SHARED((n_nodes, feat), jnp.float32),
            pltpu.VMEM_SHARED((NS, 2 * feat), jnp.float32),
            pltpu.SemaphoreType.DMA,
        ],
    )(m, dst2, src2, nn_bias, gamma, beta)


def _sc_mega1(m, dst2, nn_bias, n_nodes):
    e, feat = m.shape
    et = e // NS
    hch = et // 2 // CHUNK
    rps = n_nodes // NS
    half_n = n_nodes // NC
    orow = -(-half_n // NS)
    kk = 8

    def body(m_hbm, dsti_hbm, bias_hbm, out_hbm, idx_v, m_v, work_v, pvec,
             acc, sem):
        cid = lax.axis_index("c")
        sid = lax.axis_index("s")

        def zb(i, c0):
            work_v[i, :] = jnp.zeros((feat,), jnp.float32)
            return c0

        lax.fori_loop(0, rps, zb, 0)
        pltpu.sync_copy(work_v.at[pl.ds(0, rps)], acc.at[pl.ds(sid * rps, rps)])
        pltpu.sync_copy(bias_hbm, pvec.at[0])
        pltpu.sync_copy(dsti_hbm.at[pl.ds(sid * 2 * hch, 2 * hch)], idx_v)
        plsc.subcore_barrier()

        for half in range(2):
            pltpu.sync_copy(
                m_hbm.at[pl.ds(sid * et + half * hch * CHUNK, hch * CHUNK)], m_v)

            def grp(g, c0, _half=half):
                cb = g * kk
                handles = [
                    pltpu.async_copy(m_v.at[pl.ds((cb + j) * CHUNK, CHUNK)],
                                     acc.at[idx_v.at[_half * hch + cb + j]],
                                     sem, add=True)
                    for j in range(kk)
                ]
                for hd in handles:
                    hd.wait()
                return c0

            lax.fori_loop(0, hch // kk, grp, 0)
        plsc.subcore_barrier()

        base = cid * half_n + jnp.minimum(sid * orow, half_n - orow)
        pltpu.sync_copy(acc.at[pl.ds(base, orow)], work_v.at[pl.ds(0, orow)])
        bias_v = pvec[0, :]

        def lrow(i, c0):
            v = work_v[i, :] + bias_v
            mx = jnp.max(v)
            xm = v - mx
            ssum = jnp.sum(jnp.exp(xm))
            sv = jnp.full((feat,), ssum, jnp.float32)
            si = plsc.bitcast(sv, jnp.int32)
            expo = ((si >> 23) - 127).astype(jnp.float32)
            t = plsc.bitcast((si & 0x007FFFFF) | 0x3F800000, jnp.float32) - 1.0
            y = expo * 0.6931472 + t * (1.0 - t * (0.5 - t * 0.33333334))
            for _ in range(3):
                y = y + sv * jnp.exp(-y) - 1.0
            work_v[i, :] = xm - y
            return c0

        lax.fori_loop(0, orow, lrow, 0)
        pltpu.sync_copy(work_v.at[pl.ds(0, orow)], out_hbm.at[pl.ds(base, orow)])

    return pl.kernel(
        body,
        out_type=jax.ShapeDtypeStruct((n_nodes, feat), jnp.float32),
        mesh=_mesh(),
        compiler_params=_SC_PARAMS,
        scratch_types=[
            pltpu.VMEM((2 * hch, CHUNK), jnp.int32),
            pltpu.VMEM((hch * CHUNK, feat), jnp.float32),
            pltpu.VMEM((rps, feat), jnp.float32),
            pltpu.VMEM((1, feat), jnp.float32),
            pltpu.VMEM_SHARED((n_nodes, feat), jnp.float32),
            pltpu.SemaphoreType.DMA,
        ],
    )(m, dst2, nn_bias)


def _tc_dense(ef8, hs8, Wb, bb, S1b, S2b, rblock, real_rows):
    rows, width = ef8.shape
    kdim = S1b.shape[1]
    grid = rows // rblock

    def body(ef_ref, hs_ref, w_ref, b_ref, s1_ref, s2_ref, m_ref):
        ef = ef_ref[...]
        h = hs_ref[...]
        nf = 16
        e3 = ef.reshape(rblock, 8, nf, 1)
        h3 = h.reshape(rblock, 8, 1, nf)
        z = (jnp.broadcast_to(e3, (rblock, 8, nf, nf)) *
             jnp.broadcast_to(h3, (rblock, 8, nf, nf))
             ).reshape(rblock, 8 * nf * nf).astype(jnp.bfloat16)
        m = jnp.dot(z, w_ref[...], preferred_element_type=jnp.float32) \
            + jnp.dot(h, b_ref[...], preferred_element_type=jnp.float32)
        if real_rows != rows:
            i = pl.program_id(0)
            rr = i * rblock + lax.broadcasted_iota(jnp.int32, (rblock, width), 0)
            m = jnp.where(rr < real_rows, m, 0.0)
        m_ref[...] = m

    return pl.pallas_call(
        body,
        grid=(grid,),
        in_specs=[
            pl.BlockSpec((rblock, width), lambda i: (i, 0)),
            pl.BlockSpec((rblock, width), lambda i: (i, 0)),
            pl.BlockSpec((kdim, width), lambda i: (0, 0)),
            pl.BlockSpec((width, width), lambda i: (0, 0)),
            pl.BlockSpec((width, kdim), lambda i: (0, 0)),
            pl.BlockSpec((width, kdim), lambda i: (0, 0)),
        ],
        out_specs=pl.BlockSpec((rblock, width), lambda i: (i, 0)),
        out_shape=jax.ShapeDtypeStruct((rows, width), jnp.float32),
    )(ef8, hs8, Wb, bb, S1b, S2b)


def kernel(inputs, edge_features, edge_index, W0, b0, nn_bias0, bn_gamma0,
           bn_beta0, W1, b1, nn_bias1):
    n, nf = inputs.shape
    e = edge_features.shape[0]
    src = edge_index[0]
    dst = edge_index[1]

    gran = NW * CHUNK
    e_pad = ((e + gran - 1) // gran) * gran
    pad = e_pad - e
    if pad:
        src_p = jnp.concatenate([src, jnp.zeros((pad,), jnp.int32)])
        dst_p = jnp.concatenate([dst, jnp.zeros((pad,), jnp.int32)])
        ef_p = jnp.concatenate([edge_features,
                                jnp.zeros((pad, nf), jnp.float32)], axis=0)
    else:
        src_p, dst_p, ef_p = src, dst, edge_features
    src2 = src_p.reshape(e_pad // CHUNK, CHUNK)
    dst2 = dst_p.reshape(e_pad // CHUNK, CHUNK)

    eye = jnp.eye(nf, dtype=jnp.float32)
    S1 = jnp.repeat(eye, nf, axis=1)
    S2 = jnp.tile(eye, (1, nf))
    pk = 128 // nf
    eye8 = jnp.eye(pk, dtype=jnp.float32)
    bf = jnp.bfloat16
    S1b = jnp.kron(eye8, S1).astype(bf)
    S2b = jnp.kron(eye8, S2).astype(bf)
    Wb0 = jnp.kron(eye8, W0.reshape(nf * nf, nf)).astype(bf)
    bb0 = jnp.kron(eye8, b0.reshape(nf, nf)).astype(bf)
    Wb1 = jnp.kron(eye8, W1.reshape(nf * nf, nf)).astype(bf)
    bb1 = jnp.kron(eye8, b1.reshape(nf, nf)).astype(bf)
    ef8 = ef_p.reshape(e_pad // pk, pk * nf)
    rblock = 400
    real_rows = e // pk

    hs0 = _sc_gather(inputs, src2)
    m0 = _tc_dense(ef8, hs0.reshape(e_pad // pk, pk * nf),
                   Wb0, bb0, S1b, S2b, rblock, real_rows)
    hs1 = _sc_mega0(m0.reshape(e_pad, nf), dst2, src2,
                    nn_bias0, bn_gamma0, bn_beta0, n)

    m1 = _tc_dense(ef8, hs1.reshape(e_pad // pk, pk * nf),
                   Wb1, bb1, S1b, S2b, rblock, real_rows)
    return _sc_mega1(m1.reshape(e_pad, nf), dst2, nn_bias1, n)


# speedup vs baseline: 6.1369x; 6.1369x over previous
import jax
import jax.numpy as jnp
from jax import lax
from jax.experimental import pallas as pl
from jax.experimental.pallas import tpu as pltpu
from jax.experimental.pallas import tpu_sc as plsc

NC = 2
NS = 16
NW = NC * NS
CHUNK = 125


def _mesh():
    return plsc.VectorSubcoreMesh(
        core_axis_name="c", subcore_axis_name="s", num_cores=NC, num_subcores=NS)


_SC_PARAMS = pltpu.CompilerParams(
    use_tc_tiling_on_sc=False, needs_layout_passes=False)


def _sc_gather(table, idx2):
    n_rows = idx2.shape[0]
    ch = n_rows // NW
    epw = ch * CHUNK
    feat = table.shape[1]
    kk = 8

    def body(table_hbm, idx_hbm, out_hbm, idx_v, rows_v, sem):
        wid = lax.axis_index("c") * NS + lax.axis_index("s")
        rowbase = wid * ch
        base = wid * epw
        pltpu.sync_copy(idx_hbm.at[pl.ds(rowbase, ch)], idx_v)

        def group(g, carry):
            c0 = g * kk
            handles = [
                pltpu.async_copy(table_hbm.at[idx_v.at[c0 + j]],
                                 rows_v.at[pl.ds((c0 + j) * CHUNK, CHUNK)], sem)
                for j in range(kk)
            ]
            for hd in handles:
                hd.wait()
            return carry

        lax.fori_loop(0, ch // kk, group, 0)
        pltpu.sync_copy(rows_v, out_hbm.at[pl.ds(base, epw)])

    return pl.kernel(
        body,
        out_type=jax.ShapeDtypeStruct((n_rows * CHUNK, feat), jnp.float32),
        mesh=_mesh(),
        compiler_params=_SC_PARAMS,
        scratch_types=[
            pltpu.VMEM((ch, CHUNK), jnp.int32),
            pltpu.VMEM((epw, feat), jnp.float32),
            pltpu.SemaphoreType.DMA,
        ],
    )(table, idx2)


def _sc_mega0(m, dst2, src2, nn_bias, gamma, beta, n_nodes):
    e, feat = m.shape
    et = e // NS
    hch = et // 2 // CHUNK
    gch = e // NW // CHUNK
    gpw = gch * CHUNK
    rps = n_nodes // NS
    kk = 8
    inv_n = 1.0 / n_nodes

    def body(m_hbm, dsti_hbm, srci_hbm, bias_hbm, gam_hbm, bet_hbm,
             hs1_hbm, idx_v, m_v, work_v, stats_my, stats_all, pvec,
             acc, stats_sh, sem):
        cid = lax.axis_index("c")
        sid = lax.axis_index("s")
        wid = cid * NS + sid

        def zb(i, c0):
            work_v[i, :] = jnp.zeros((feat,), jnp.float32)
            return c0

        lax.fori_loop(0, rps, zb, 0)
        pltpu.sync_copy(work_v, acc.at[pl.ds(sid * rps, rps)])
        pltpu.sync_copy(bias_hbm, pvec.at[0])
        pltpu.sync_copy(gam_hbm, pvec.at[1])
        pltpu.sync_copy(bet_hbm, pvec.at[2])
        pltpu.sync_copy(dsti_hbm.at[pl.ds(sid * 2 * hch, 2 * hch)], idx_v)
        plsc.subcore_barrier()

        for half in range(2):
            pltpu.sync_copy(
                m_hbm.at[pl.ds(sid * et + half * hch * CHUNK, hch * CHUNK)], m_v)

            def grp(g, c0, _half=half):
                cb = g * kk
                handles = [
                    pltpu.async_copy(m_v.at[pl.ds((cb + j) * CHUNK, CHUNK)],
                                     acc.at[idx_v.at[_half * hch + cb + j]],
                                     sem, add=True)
                    for j in range(kk)
                ]
                for hd in handles:
                    hd.wait()
                return c0

            lax.fori_loop(0, hch // kk, grp, 0)
        plsc.subcore_barrier()

        pltpu.sync_copy(acc.at[pl.ds(sid * rps, rps)], work_v)
        bias_v = pvec[0, :]
        gam_v = pvec[1, :]
        bet_v = pvec[2, :]

        def bn1(i, carry):
            s, ss = carry
            v = jnp.maximum(work_v[i, :] + bias_v, 0.0)
            work_v[i, :] = v
            return (s + v, ss + v * v)

        zv = jnp.zeros((feat,), jnp.float32)
        s, ss = lax.fori_loop(0, rps, bn1, (zv, zv))
        stats_my[pl.ds(0, feat)] = s
        stats_my[pl.ds(feat, feat)] = ss
        pltpu.sync_copy(stats_my, stats_sh.at[sid])
        plsc.subcore_barrier()

        pltpu.sync_copy(stats_sh, stats_all)
        tot = jnp.zeros((feat,), jnp.float32)
        tot2 = jnp.zeros((feat,), jnp.float32)
        for t in range(NS):
            tot = tot + stats_all[t, pl.ds(0, feat)]
            tot2 = tot2 + stats_all[t, pl.ds(feat, feat)]
        mean = tot * inv_n
        var = tot2 * inv_n - mean * mean
        x = var + 1e-5
        yi = jnp.int32(0x5F3759DF) - (plsc.bitcast(x, jnp.int32) >> 1)
        y = plsc.bitcast(yi, jnp.float32)
        for _ in range(3):
            y = y * (1.5 - 0.5 * x * y * y)
        scale = gam_v * y
        shift = bet_v - mean * scale

        def bn2(i, c0):
            work_v[i, :] = work_v[i, :] * scale + shift
            return c0

        lax.fori_loop(0, rps, bn2, 0)
        pltpu.sync_copy(work_v, acc.at[pl.ds(sid * rps, rps)])
        plsc.subcore_barrier()

        pltpu.sync_copy(srci_hbm.at[pl.ds(wid * gch, gch)],
                        idx_v.at[pl.ds(0, gch)])

        def ggrp(g, c0):
            cb = g * kk
            handles = [
                pltpu.async_copy(acc.at[idx_v.at[cb + j]],
                                 m_v.at[pl.ds((cb + j) * CHUNK, CHUNK)], sem)
                for j in range(kk)
            ]
            for hd in handles:
                hd.wait()
            return c0

        lax.fori_loop(0, gch // kk, ggrp, 0)
        pltpu.sync_copy(m_v, hs1_hbm.at[pl.ds(wid * gpw, gpw)])

    return pl.kernel(
        body,
        out_type=jax.ShapeDtypeStruct((e, feat), jnp.float32),
        mesh=_mesh(),
        compiler_params=_SC_PARAMS,
        scratch_types=[
            pltpu.VMEM((2 * hch, CHUNK), jnp.int32),
            pltpu.VMEM((hch * CHUNK, feat), jnp.float32),
            pltpu.VMEM((rps, feat), jnp.float32),
            pltpu.VMEM((2 * feat,), jnp.float32),
            pltpu.VMEM((NS, 2 * feat), jnp.float32),
            pltpu.VMEM((3, feat), jnp.float32),
            pltpu.VMEM_SHARED((n_nodes, feat), jnp.float32),
            pltpu.VMEM_SHARED((NS, 2 * feat), jnp.float32),
            pltpu.SemaphoreType.DMA,
        ],
    )(m, dst2, src2, nn_bias, gamma, beta)


def _sc_mega1(m, dst2, nn_bias, n_nodes):
    e, feat = m.shape
    et = e // NS
    hch = et // 2 // CHUNK
    rps = n_nodes // NS
    half_n = n_nodes // NC
    orow = -(-half_n // NS)
    kk = 8

    def body(m_hbm, dsti_hbm, bias_hbm, out_hbm, idx_v, m_v, work_v, pvec,
             acc, sem):
        cid = lax.axis_index("c")
        sid = lax.axis_index("s")

        def zb(i, c0):
            work_v[i, :] = jnp.zeros((feat,), jnp.float32)
            return c0

        lax.fori_loop(0, rps, zb, 0)
        pltpu.sync_copy(work_v.at[pl.ds(0, rps)], acc.at[pl.ds(sid * rps, rps)])
        pltpu.sync_copy(bias_hbm, pvec.at[0])
        pltpu.sync_copy(dsti_hbm.at[pl.ds(sid * 2 * hch, 2 * hch)], idx_v)
        plsc.subcore_barrier()

        for half in range(2):
            pltpu.sync_copy(
                m_hbm.at[pl.ds(sid * et + half * hch * CHUNK, hch * CHUNK)], m_v)

            def grp(g, c0, _half=half):
                cb = g * kk
                handles = [
                    pltpu.async_copy(m_v.at[pl.ds((cb + j) * CHUNK, CHUNK)],
                                     acc.at[idx_v.at[_half * hch + cb + j]],
                                     sem, add=True)
                    for j in range(kk)
                ]
                for hd in handles:
                    hd.wait()
                return c0

            lax.fori_loop(0, hch // kk, grp, 0)
        plsc.subcore_barrier()

        base = cid * half_n + jnp.minimum(sid * orow, half_n - orow)
        pltpu.sync_copy(acc.at[pl.ds(base, orow)], work_v.at[pl.ds(0, orow)])
        bias_v = pvec[0, :]

        def lrow(i, c0):
            v = work_v[i, :] + bias_v
            mx = jnp.max(v)
            xm = v - mx
            ssum = jnp.sum(jnp.exp(xm))
            sv = jnp.full((feat,), ssum, jnp.float32)
            si = plsc.bitcast(sv, jnp.int32)
            expo = ((si >> 23) - 127).astype(jnp.float32)
            t = plsc.bitcast((si & 0x007FFFFF) | 0x3F800000, jnp.float32) - 1.0
            y = expo * 0.6931472 + t * (1.0 - t * (0.5 - t * 0.33333334))
            for _ in range(3):
                y = y + sv * jnp.exp(-y) - 1.0
            work_v[i, :] = xm - y
            return c0

        lax.fori_loop(0, orow, lrow, 0)
        pltpu.sync_copy(work_v.at[pl.ds(0, orow)], out_hbm.at[pl.ds(base, orow)])

    return pl.kernel(
        body,
        out_type=jax.ShapeDtypeStruct((n_nodes, feat), jnp.float32),
        mesh=_mesh(),
        compiler_params=_SC_PARAMS,
        scratch_types=[
            pltpu.VMEM((2 * hch, CHUNK), jnp.int32),
            pltpu.VMEM((hch * CHUNK, feat), jnp.float32),
            pltpu.VMEM((rps, feat), jnp.float32),
            pltpu.VMEM((1, feat), jnp.float32),
            pltpu.VMEM_SHARED((n_nodes, feat), jnp.float32),
            pltpu.SemaphoreType.DMA,
        ],
    )(m, dst2, nn_bias)


def _tc_dense(ef8, hs8, Wb, bb, S1b, S2b, rblock, real_rows):
    rows, width = ef8.shape
    kdim = S1b.shape[1]
    grid = rows // rblock

    def body(ef_ref, hs_ref, w_ref, b_ref, s1_ref, s2_ref, m_ref):
        ef = ef_ref[...].astype(jnp.bfloat16)
        hb = hs_ref[...].astype(jnp.bfloat16)
        a = jnp.dot(ef, s1_ref[...], preferred_element_type=jnp.float32)
        c = jnp.dot(hb, s2_ref[...], preferred_element_type=jnp.float32)
        z = (a * c).astype(jnp.bfloat16)
        m = jnp.dot(z, w_ref[...], preferred_element_type=jnp.float32) \
            + jnp.dot(hb, b_ref[...], preferred_element_type=jnp.float32)
        if real_rows != rows:
            i = pl.program_id(0)
            rr = i * rblock + lax.broadcasted_iota(jnp.int32, (rblock, width), 0)
            m = jnp.where(rr < real_rows, m, 0.0)
        m_ref[...] = m

    return pl.pallas_call(
        body,
        grid=(grid,),
        in_specs=[
            pl.BlockSpec((rblock, width), lambda i: (i, 0)),
            pl.BlockSpec((rblock, width), lambda i: (i, 0)),
            pl.BlockSpec((kdim, width), lambda i: (0, 0)),
            pl.BlockSpec((width, width), lambda i: (0, 0)),
            pl.BlockSpec((width, kdim), lambda i: (0, 0)),
            pl.BlockSpec((width, kdim), lambda i: (0, 0)),
        ],
        out_specs=pl.BlockSpec((rblock, width), lambda i: (i, 0)),
        out_shape=jax.ShapeDtypeStruct((rows, width), jnp.float32),
    )(ef8, hs8, Wb, bb, S1b, S2b)


def kernel(inputs, edge_features, edge_index, W0, b0, nn_bias0, bn_gamma0,
           bn_beta0, W1, b1, nn_bias1):
    n, nf = inputs.shape
    e = edge_features.shape[0]
    src = edge_index[0]
    dst = edge_index[1]

    gran = NW * CHUNK
    e_pad = ((e + gran - 1) // gran) * gran
    pad = e_pad - e
    if pad:
        src_p = jnp.concatenate([src, jnp.zeros((pad,), jnp.int32)])
        dst_p = jnp.concatenate([dst, jnp.zeros((pad,), jnp.int32)])
        ef_p = jnp.concatenate([edge_features,
                                jnp.zeros((pad, nf), jnp.float32)], axis=0)
    else:
        src_p, dst_p, ef_p = src, dst, edge_features
    src2 = src_p.reshape(e_pad // CHUNK, CHUNK)
    dst2 = dst_p.reshape(e_pad // CHUNK, CHUNK)

    eye = jnp.eye(nf, dtype=jnp.float32)
    S1 = jnp.repeat(eye, nf, axis=1)
    S2 = jnp.tile(eye, (1, nf))
    pk = 128 // nf
    eye8 = jnp.eye(pk, dtype=jnp.float32)
    bf = jnp.bfloat16
    S1b = jnp.kron(eye8, S1).astype(bf)
    S2b = jnp.kron(eye8, S2).astype(bf)
    Wb0 = jnp.kron(eye8, W0.reshape(nf * nf, nf)).astype(bf)
    bb0 = jnp.kron(eye8, b0.reshape(nf, nf)).astype(bf)
    Wb1 = jnp.kron(eye8, W1.reshape(nf * nf, nf)).astype(bf)
    bb1 = jnp.kron(eye8, b1.reshape(nf, nf)).astype(bf)
    ef8 = ef_p.reshape(e_pad // pk, pk * nf)
    rblock = 1000
    real_rows = e // pk

    hs0 = _sc_gather(inputs, src2)
    m0 = _tc_dense(ef8, hs0.reshape(e_pad // pk, pk * nf),
                   Wb0, bb0, S1b, S2b, rblock, real_rows)
    hs1 = _sc_mega0(m0.reshape(e_pad, nf), dst2, src2,
                    nn_bias0, bn_gamma0, bn_beta0, n)

    m1 = _tc_dense(ef8, hs1.reshape(e_pad // pk, pk * nf),
                   Wb1, bb1, S1b, S2b, rblock, real_rows)
    return _sc_mega1(m1.reshape(e_pad, nf), dst2, nn_bias1, n)


# speedup vs baseline: 6.2305x; 1.0153x over previous
import jax
import jax.numpy as jnp
from jax import lax
from jax.experimental import pallas as pl
from jax.experimental.pallas import tpu as pltpu
from jax.experimental.pallas import tpu_sc as plsc

NC = 2
NS = 16
NW = NC * NS
CHUNK = 125


def _mesh():
    return plsc.VectorSubcoreMesh(
        core_axis_name="c", subcore_axis_name="s", num_cores=NC, num_subcores=NS)


_SC_PARAMS = pltpu.CompilerParams(
    use_tc_tiling_on_sc=False, needs_layout_passes=False)


def _sc_gather(table, idx2):
    n_rows = idx2.shape[0]
    ch = n_rows // NW
    epw = ch * CHUNK
    feat = table.shape[1]
    kk = 8

    def body(table_hbm, idx_hbm, out_hbm, idx_v, rows_v, sem):
        wid = lax.axis_index("c") * NS + lax.axis_index("s")
        rowbase = wid * ch
        base = wid * epw
        pltpu.sync_copy(idx_hbm.at[pl.ds(rowbase, ch)], idx_v)

        def group(g, carry):
            c0 = g * kk
            handles = [
                pltpu.async_copy(table_hbm.at[idx_v.at[c0 + j]],
                                 rows_v.at[pl.ds((c0 + j) * CHUNK, CHUNK)], sem)
                for j in range(kk)
            ]
            for hd in handles:
                hd.wait()
            return carry

        lax.fori_loop(0, ch // kk, group, 0)
        pltpu.sync_copy(rows_v, out_hbm.at[pl.ds(base, epw)])

    return pl.kernel(
        body,
        out_type=jax.ShapeDtypeStruct((n_rows * CHUNK, feat), jnp.float32),
        mesh=_mesh(),
        compiler_params=_SC_PARAMS,
        scratch_types=[
            pltpu.VMEM((ch, CHUNK), jnp.int32),
            pltpu.VMEM((epw, feat), jnp.float32),
            pltpu.SemaphoreType.DMA,
        ],
    )(table, idx2)


def _sc_mega0(m, dst2, src2, nn_bias, gamma, beta, n_nodes):
    e, feat = m.shape
    et = e // NS
    hch = et // 2 // CHUNK
    gch = e // NW // CHUNK
    gpw = gch * CHUNK
    rps = n_nodes // NS
    kk = 8
    inv_n = 1.0 / n_nodes

    def body(m_hbm, dsti_hbm, srci_hbm, bias_hbm, gam_hbm, bet_hbm,
             hs1_hbm, idx_v, m_v, work_v, stats_my, stats_all, pvec,
             acc, stats_sh, sem):
        cid = lax.axis_index("c")
        sid = lax.axis_index("s")
        wid = cid * NS + sid

        def zb(i, c0):
            work_v[i, :] = jnp.zeros((feat,), jnp.float32)
            return c0

        lax.fori_loop(0, rps, zb, 0)
        pltpu.sync_copy(work_v, acc.at[pl.ds(sid * rps, rps)])
        pltpu.sync_copy(bias_hbm, pvec.at[0])
        pltpu.sync_copy(gam_hbm, pvec.at[1])
        pltpu.sync_copy(bet_hbm, pvec.at[2])
        pltpu.sync_copy(dsti_hbm.at[pl.ds(sid * 2 * hch, 2 * hch)], idx_v)
        plsc.subcore_barrier()

        for half in range(2):
            pltpu.sync_copy(
                m_hbm.at[pl.ds(sid * et + half * hch * CHUNK, hch * CHUNK)], m_v)

            def grp(g, c0, _half=half):
                cb = g * kk
                handles = [
                    pltpu.async_copy(m_v.at[pl.ds((cb + j) * CHUNK, CHUNK)],
                                     acc.at[idx_v.at[_half * hch + cb + j]],
                                     sem, add=True)
                    for j in range(kk)
                ]
                for hd in handles:
                    hd.wait()
                return c0

            lax.fori_loop(0, hch // kk, grp, 0)
        plsc.subcore_barrier()

        pltpu.sync_copy(acc.at[pl.ds(sid * rps, rps)], work_v)
        bias_v = pvec[0, :]
        gam_v = pvec[1, :]
        bet_v = pvec[2, :]

        def bn1(i, carry):
            s, ss = carry
            v = jnp.maximum(work_v[i, :] + bias_v, 0.0)
            work_v[i, :] = v
            return (s + v, ss + v * v)

        zv = jnp.zeros((feat,), jnp.float32)
        s, ss = lax.fori_loop(0, rps, bn1, (zv, zv))
        stats_my[pl.ds(0, feat)] = s
        stats_my[pl.ds(feat, feat)] = ss
        pltpu.sync_copy(stats_my, stats_sh.at[sid])
        plsc.subcore_barrier()

        pltpu.sync_copy(stats_sh, stats_all)
        tot = jnp.zeros((feat,), jnp.float32)
        tot2 = jnp.zeros((feat,), jnp.float32)
        for t in range(NS):
            tot = tot + stats_all[t, pl.ds(0, feat)]
            tot2 = tot2 + stats_all[t, pl.ds(feat, feat)]
        mean = tot * inv_n
        var = tot2 * inv_n - mean * mean
        x = var + 1e-5
        yi = jnp.int32(0x5F3759DF) - (plsc.bitcast(x, jnp.int32) >> 1)
        y = plsc.bitcast(yi, jnp.float32)
        for _ in range(3):
            y = y * (1.5 - 0.5 * x * y * y)
        scale = gam_v * y
        shift = bet_v - mean * scale

        def bn2(i, c0):
            work_v[i, :] = work_v[i, :] * scale + shift
            return c0

        lax.fori_loop(0, rps, bn2, 0)
        pltpu.sync_copy(work_v, acc.at[pl.ds(sid * rps, rps)])
        plsc.subcore_barrier()

        pltpu.sync_copy(srci_hbm.at[pl.ds(wid * gch, gch)],
                        idx_v.at[pl.ds(0, gch)])

        def ggrp(g, c0):
            cb = g * kk
            handles = [
                pltpu.async_copy(acc.at[idx_v.at[cb + j]],
                                 m_v.at[pl.ds((cb + j) * CHUNK, CHUNK)], sem)
                for j in range(kk)
            ]
            for hd in handles:
                hd.wait()
            return c0

        lax.fori_loop(0, gch // kk, ggrp, 0)
        pltpu.sync_copy(m_v, hs1_hbm.at[pl.ds(wid * gpw, gpw)])

    return pl.kernel(
        body,
        out_type=jax.ShapeDtypeStruct((e, feat), jnp.float32),
        mesh=_mesh(),
        compiler_params=_SC_PARAMS,
        scratch_types=[
            pltpu.VMEM((2 * hch, CHUNK), jnp.int32),
            pltpu.VMEM((hch * CHUNK, feat), jnp.float32),
            pltpu.VMEM((rps, feat), jnp.float32),
            pltpu.VMEM((2 * feat,), jnp.float32),
            pltpu.VMEM((NS, 2 * feat), jnp.float32),
            pltpu.VMEM((3, feat), jnp.float32),
            pltpu.VMEM_SHARED((n_nodes, feat), jnp.float32),
            pltpu.VMEM_SHARED((NS, 2 * feat), jnp.float32),
            pltpu.SemaphoreType.DMA,
        ],
    )(m, dst2, src2, nn_bias, gamma, beta)


def _sc_mega1(m, dst2, nn_bias, n_nodes):
    e, feat = m.shape
    et = e // NS
    hch = et // 2 // CHUNK
    rps = n_nodes // NS
    half_n = n_nodes // NC
    orow = -(-half_n // NS)
    kk = 8

    def body(m_hbm, dsti_hbm, bias_hbm, out_hbm, idx_v, m_v, work_v, pvec,
             acc, sem):
        cid = lax.axis_index("c")
        sid = lax.axis_index("s")

        def zb(i, c0):
            work_v[i, :] = jnp.zeros((feat,), jnp.float32)
            return c0

        lax.fori_loop(0, rps, zb, 0)
        pltpu.sync_copy(work_v.at[pl.ds(0, rps)], acc.at[pl.ds(sid * rps, rps)])
        pltpu.sync_copy(bias_hbm, pvec.at[0])
        pltpu.sync_copy(dsti_hbm.at[pl.ds(sid * 2 * hch, 2 * hch)], idx_v)
        plsc.subcore_barrier()

        for half in range(2):
            pltpu.sync_copy(
                m_hbm.at[pl.ds(sid * et + half * hch * CHUNK, hch * CHUNK)], m_v)

            def grp(g, c0, _half=half):
                cb = g * kk
                handles = [
                    pltpu.async_copy(m_v.at[pl.ds((cb + j) * CHUNK, CHUNK)],
                                     acc.at[idx_v.at[_half * hch + cb + j]],
                                     sem, add=True)
                    for j in range(kk)
                ]
                for hd in handles:
                    hd.wait()
                return c0

            lax.fori_loop(0, hch // kk, grp, 0)
        plsc.subcore_barrier()

        base = cid * half_n + jnp.minimum(sid * orow, half_n - orow)
        pltpu.sync_copy(acc.at[pl.ds(base, orow)], work_v.at[pl.ds(0, orow)])
        bias_v = pvec[0, :]

        def lrow(i, c0):
            v = work_v[i, :] + bias_v
            mx = jnp.max(v)
            xm = v - mx
            ssum = jnp.sum(jnp.exp(xm))
            sv = jnp.full((feat,), ssum, jnp.float32)
            si = plsc.bitcast(sv, jnp.int32)
            expo = ((si >> 23) - 127).astype(jnp.float32)
            t = plsc.bitcast((si & 0x007FFFFF) | 0x3F800000, jnp.float32) - 1.0
            y = expo * 0.6931472 + t * (1.0 - t * (0.5 - t * 0.33333334))
            for _ in range(3):
                y = y + sv * jnp.exp(-y) - 1.0
            work_v[i, :] = xm - y
            return c0

        lax.fori_loop(0, orow, lrow, 0)
        pltpu.sync_copy(work_v.at[pl.ds(0, orow)], out_hbm.at[pl.ds(base, orow)])

    return pl.kernel(
        body,
        out_type=jax.ShapeDtypeStruct((n_nodes, feat), jnp.float32),
        mesh=_mesh(),
        compiler_params=_SC_PARAMS,
        scratch_types=[
            pltpu.VMEM((2 * hch, CHUNK), jnp.int32),
            pltpu.VMEM((hch * CHUNK, feat), jnp.float32),
            pltpu.VMEM((rps, feat), jnp.float32),
            pltpu.VMEM((1, feat), jnp.float32),
            pltpu.VMEM_SHARED((n_nodes, feat), jnp.float32),
            pltpu.SemaphoreType.DMA,
        ],
    )(m, dst2, nn_bias)


def _tc_dense(ef8, hs8, Wb, bb, S1b, S2b, rblock, real_rows):
    rows, width = ef8.shape
    kdim = S1b.shape[1]
    grid = rows // rblock

    def body(ef_ref, hs_ref, w_ref, b_ref, s1_ref, s2_ref, m_ref):
        ef = ef_ref[...].astype(jnp.bfloat16)
        hb = hs_ref[...].astype(jnp.bfloat16)
        a = jnp.dot(ef, s1_ref[...], preferred_element_type=jnp.float32)
        c = jnp.dot(hb, s2_ref[...], preferred_element_type=jnp.float32)
        z = (a * c).astype(jnp.bfloat16)
        m = jnp.dot(z, w_ref[...], preferred_element_type=jnp.float32) \
            + jnp.dot(hb, b_ref[...], preferred_element_type=jnp.float32)
        if real_rows != rows:
            i = pl.program_id(0)
            rr = i * rblock + lax.broadcasted_iota(jnp.int32, (rblock, width), 0)
            m = jnp.where(rr < real_rows, m, 0.0)
        m_ref[...] = m

    return pl.pallas_call(
        body,
        grid=(grid,),
        in_specs=[
            pl.BlockSpec((rblock, width), lambda i: (i, 0)),
            pl.BlockSpec((rblock, width), lambda i: (i, 0)),
            pl.BlockSpec((kdim, width), lambda i: (0, 0)),
            pl.BlockSpec((width, width), lambda i: (0, 0)),
            pl.BlockSpec((width, kdim), lambda i: (0, 0)),
            pl.BlockSpec((width, kdim), lambda i: (0, 0)),
        ],
        out_specs=pl.BlockSpec((rblock, width), lambda i: (i, 0)),
        out_shape=jax.ShapeDtypeStruct((rows, width), jnp.float32),
    )(ef8, hs8, Wb, bb, S1b, S2b)


def kernel(inputs, edge_features, edge_index, W0, b0, nn_bias0, bn_gamma0,
           bn_beta0, W1, b1, nn_bias1):
    n, nf = inputs.shape
    e = edge_features.shape[0]
    src = edge_index[0]
    dst = edge_index[1]

    gran = NW * CHUNK
    e_pad = ((e + gran - 1) // gran) * gran
    pad = e_pad - e
    if pad:
        src_p = jnp.concatenate([src, jnp.zeros((pad,), jnp.int32)])
        dst_p = jnp.concatenate([dst, jnp.zeros((pad,), jnp.int32)])
        ef_p = jnp.concatenate([edge_features,
                                jnp.zeros((pad, nf), jnp.float32)], axis=0)
    else:
        src_p, dst_p, ef_p = src, dst, edge_features
    src2 = src_p.reshape(e_pad // CHUNK, CHUNK)
    dst2 = dst_p.reshape(e_pad // CHUNK, CHUNK)

    eye = jnp.eye(nf, dtype=jnp.float32)
    S1 = jnp.repeat(eye, nf, axis=1)
    S2 = jnp.tile(eye, (1, nf))
    pk = 128 // nf
    eye8 = jnp.eye(pk, dtype=jnp.float32)
    bf = jnp.bfloat16
    S1b = jnp.kron(eye8, S1).astype(bf)
    S2b = jnp.kron(eye8, S2).astype(bf)
    Wb0 = jnp.kron(eye8, W0.reshape(nf * nf, nf)).astype(bf)
    bb0 = jnp.kron(eye8, b0.reshape(nf, nf)).astype(bf)
    Wb1 = jnp.kron(eye8, W1.reshape(nf * nf, nf)).astype(bf)
    bb1 = jnp.kron(eye8, b1.reshape(nf, nf)).astype(bf)
    ef8 = ef_p.reshape(e_pad // pk, pk * nf)
    rblock = 2000
    real_rows = e // pk

    hs0 = _sc_gather(inputs, src2)
    m0 = _tc_dense(ef8, hs0.reshape(e_pad // pk, pk * nf),
                   Wb0, bb0, S1b, S2b, rblock, real_rows)
    hs1 = _sc_mega0(m0.reshape(e_pad, nf), dst2, src2,
                    nn_bias0, bn_gamma0, bn_beta0, n)

    m1 = _tc_dense(ef8, hs1.reshape(e_pad // pk, pk * nf),
                   Wb1, bb1, S1b, S2b, rblock, real_rows)
    return _sc_mega1(m1.reshape(e_pad, nf), dst2, nn_bias1, n)
